# baseline (speedup 1.0000x reference)
import functools

import jax
import jax.numpy as jnp
from jax import lax
from jax.experimental import pallas as pl
from jax.experimental.pallas import tpu as pltpu
from jax.experimental.pallas import tpu_sc as plsc

N = 10000
E = 320000
D = 128
TW = 128
OW = 128
CB = 128
NWORK = 32
ZB = 80
NZB = N // ZB
NSLAB = 4
SL = E // NSLAB


def _silu(x):
    t = 0.5 * x
    return t * (jnp.tanh(t) + 1.0)


def _silu_b(x):
    xb = x.astype(jnp.bfloat16)
    return xb * jax.nn.sigmoid(xb)


def _dot_t(x, w):
    return lax.dot_general(x, w, (((1,), (1,)), ((), ())),
                           preferred_element_type=jnp.float32)


def _dot_t_bf(x, w):
    return lax.dot_general(x.astype(jnp.bfloat16), w.astype(jnp.bfloat16),
                           (((1,), (1,)), ((), ())),
                           preferred_element_type=jnp.float32)


def _tc_pre_body(h_ref, c_ref, a_ref, b_ref, ta_ref, tb_ref):
    h = h_ref[...]
    cd = c_ref[...]
    pad = jnp.zeros((h.shape[0], TW - 67), jnp.float32)
    ha = _dot_t(h, a_ref[...])
    hb = _dot_t(h, b_ref[...])
    ta_ref[...] = jnp.concatenate([ha, cd, pad], axis=1)
    tb_ref[...] = jnp.concatenate([hb, -cd, pad], axis=1)


def _tc_pre(h, coord, a, b):
    blk = 2000
    grid = N // blk
    return pl.pallas_call(
        _tc_pre_body,
        grid=(grid,),
        in_specs=[
            pl.BlockSpec((blk, D), lambda i: (i, 0)),
            pl.BlockSpec((blk, 3), lambda i: (i, 0)),
            pl.BlockSpec((64, D), lambda i: (0, 0)),
            pl.BlockSpec((64, D), lambda i: (0, 0)),
        ],
        out_specs=[
            pl.BlockSpec((blk, TW), lambda i: (i, 0)),
            pl.BlockSpec((blk, TW), lambda i: (i, 0)),
        ],
        out_shape=[
            jax.ShapeDtypeStruct((N, TW), jnp.float32),
            jax.ShapeDtypeStruct((N, TW), jnp.float32),
        ],
    )(h, coord, a, b)


GCB = 128
GW = 128


def _sc_gather(ta, tb, rcf):
    mesh = plsc.VectorSubcoreMesh(core_axis_name="c", subcore_axis_name="s")

    ne = rcf.shape[0] // 2
    nchunk = ne // GCB
    nit = (nchunk + NWORK - 1) // NWORK + 2

    @functools.partial(
        pl.kernel,
        out_type=jax.ShapeDtypeStruct((2, ne, GW), jnp.float32),
        mesh=mesh,
        compiler_params=pltpu.CompilerParams(use_tc_tiling_on_sc=True),
        scratch_types=[
            pltpu.VMEM((2, 2, GCB), jnp.int32),
            pltpu.VMEM((2, 2, GCB, TW), jnp.float32),
            pltpu.SemaphoreType.DMA,
            pltpu.SemaphoreType.DMA,
            pltpu.SemaphoreType.DMA,
            pltpu.SemaphoreType.DMA,
            pltpu.SemaphoreType.DMA,
            pltpu.SemaphoreType.DMA,
        ],
    )
    def k(ta_hbm, tb_hbm, rcf_hbm, gab_hbm,
          idx, buf, isem0, isem1, gsem0, gsem1, wsem0, wsem1):
        cid = lax.axis_index("c")
        sid = lax.axis_index("s")
        wid = sid * 2 + cid
        isems = (isem0, isem1)
        gsems = (gsem0, gsem1)
        wsems = (wsem0, wsem1)

        def chunk_of(t):
            return wid + t * NWORK

        def valid(t):
            return jnp.logical_and(t >= 0, chunk_of(t) < nchunk)

        def issue_idx(t, slot):
            base = pl.multiple_of(chunk_of(t) * GCB, 8)
            pltpu.async_copy(rcf_hbm.at[pl.ds(base, GCB)], idx.at[slot, 0],
                             isems[slot])
            pltpu.async_copy(rcf_hbm.at[pl.ds(ne + base, GCB)],
                             idx.at[slot, 1], isems[slot])

        def issue_gather(slot):
            pltpu.async_copy(ta_hbm.at[idx.at[slot, 0]], buf.at[slot, 0],
                             gsems[slot])
            pltpu.async_copy(tb_hbm.at[idx.at[slot, 1]], buf.at[slot, 1],
                             gsems[slot])

        def wait(sem, dst):
            pltpu.make_async_copy(ta_hbm.at[pl.ds(0, dst.shape[-2])], dst,
                                  sem).wait()

        @pl.when(valid(0))
        def _():
            issue_idx(0, 0)

        @pl.loop(0, (nit + 1) // 2)
        def _(t2):
            for b in range(2):
                slot, nslot = b, 1 - b
                t = t2 * 2 + b

                @pl.when(valid(t - 2))
                def _():
                    pltpu.make_async_copy(buf.at[slot, :, :, pl.ds(0, GW)],
                                          gab_hbm.at[:, pl.ds(0, GCB)],
                                          wsems[slot]).wait()

                @pl.when(valid(t))
                def _():
                    pltpu.make_async_copy(rcf_hbm.at[pl.ds(0, GCB)],
                                          idx.at[slot, 0], isems[slot]).wait()
                    pltpu.make_async_copy(rcf_hbm.at[pl.ds(0, GCB)],
                                          idx.at[slot, 1], isems[slot]).wait()
                    issue_gather(slot)

                @pl.when(valid(t - 1))
                def _():
                    wait(gsems[nslot], buf.at[nslot, 0])
                    wait(gsems[nslot], buf.at[nslot, 1])
                    base = pl.multiple_of(chunk_of(t - 1) * GCB, 8)
                    pltpu.async_copy(buf.at[nslot, :, :, pl.ds(0, GW)],
                                     gab_hbm.at[:, pl.ds(base, GCB)],
                                     wsems[nslot])

                @pl.when(valid(t + 1))
                def _():
                    issue_idx(t + 1, nslot)

    return k(ta, tb, rcf)


def _tc_edge_body(ga_ref, gb_ref, ea_ref,
                  wr_ref, c4_ref, eb0_ref, ew1_ref, eb1_ref, ew2_ref, eb2_ref,
                  cw0_ref, cb0_ref, cw1_ref, cb1_ref, cw2_ref, nw0r_ref,
                  out_ref):
    s = ga_ref[0] + gb_ref[0]
    z0p = s[:, :64]
    cdiff = s[:, 64:67]
    radial = jnp.sum(cdiff * cdiff, axis=1, keepdims=True)
    ea = ea_ref[...]
    z0 = z0p + radial * wr_ref[...] + jnp.dot(
        ea, c4_ref[...], preferred_element_type=jnp.float32) + eb0_ref[...]
    z0 = _silu(z0)
    z1 = _silu(_dot_t_bf(z0, ew1_ref[...]) + eb1_ref[...])
    ef = _silu(_dot_t_bf(z1, ew2_ref[...]) + eb2_ref[...])
    c0 = _silu(_dot_t_bf(ef, cw0_ref[...]) + cb0_ref[...])
    c1 = _silu(_dot_t_bf(c0, cw1_ref[...]) + cb1_ref[...])
    cc = jnp.sum(c1 * cw2_ref[...], axis=1, keepdims=True)
    trans = jnp.clip(cdiff * cc, -100.0, 100.0)
    q = _dot_t_bf(ef, nw0r_ref[...])
    nb = s.shape[0]
    out_ref[:, 0:64] = q
    out_ref[:, 64:67] = trans
    out_ref[:, 67:68] = jnp.ones((nb, 1), jnp.float32)


def _tc_edge(gab, ea, wr, c4, eb0, ew1, eb1, ew2, eb2,
             cw0, cb0, cw1, cb1, cw2, nw0r):
    blk = 3200
    ne = gab.shape[1]
    grid = ne // blk
    full = lambda shp: pl.BlockSpec(shp, lambda i: tuple(0 for _ in shp))
    return pl.pallas_call(
        _tc_edge_body,
        grid=(grid,),
        in_specs=[
            pl.BlockSpec((1, blk, GW), lambda i: (0, i, 0)),
            pl.BlockSpec((1, blk, GW), lambda i: (1, i, 0)),
            pl.BlockSpec((blk, 4), lambda i: (i, 0)),
            full(wr.shape), full(c4.shape), full(eb0.shape),
            full(ew1.shape), full(eb1.shape), full(ew2.shape), full(eb2.shape),
            full(cw0.shape), full(cb0.shape), full(cw1.shape), full(cb1.shape),
            full(cw2.shape), full(nw0r.shape),
        ],
        out_specs=pl.BlockSpec((blk, OW), lambda i: (i, 0)),
        out_shape=jax.ShapeDtypeStruct((ne, OW), jnp.float32),
    )(gab, gab, ea, wr, c4, eb0, ew1, eb1, ew2, eb2, cw0, cb0, cw1, cb1, cw2,
      nw0r)


def _sc_scatter(out_e, row):
    mesh = plsc.VectorSubcoreMesh(core_axis_name="c", subcore_axis_name="s")
    nchunk = row.shape[0] // CB
    maxk = (nchunk + 1) // 2

    @functools.partial(
        pl.kernel,
        out_type=jax.ShapeDtypeStruct((2, N, OW), jnp.float32),
        mesh=mesh,
        compiler_params=pltpu.CompilerParams(use_tc_tiling_on_sc=True),
        scratch_types=[
            pltpu.VMEM((2, CB), jnp.int32),
            pltpu.VMEM((2, CB, OW), jnp.float32),
            pltpu.VMEM((ZB, OW), jnp.float32),
            pltpu.VMEM_SHARED((N, OW), jnp.float32),
            pltpu.SemaphoreType.DMA,
            pltpu.SemaphoreType.DMA,
        ],
    )
    def k(oe_hbm, row_hbm, p_hbm, idx, buf, zbuf, acc, lsem0, lsem1):
        cid = lax.axis_index("c")
        sid = lax.axis_index("s")
        lsems = (lsem0, lsem1)

        @pl.loop(0, ZB)
        def _(i):
            @pl.loop(0, OW // 16)
            def _(j):
                zbuf[i, pl.ds(j * 16, 16)] = jnp.zeros((16,), jnp.float32)

        @pl.loop(0, (NZB + 15) // 16)
        def _(t):
            blkid = sid + t * 16

            @pl.when(blkid < NZB)
            def _():
                off = pl.multiple_of(blkid * ZB, 8)
                pltpu.sync_copy(zbuf, acc.at[pl.ds(off, ZB)])

        plsc.subcore_barrier()

        def ch_of(t):
            return (sid + t * 16) * 2 + cid

        def issue_loads(t, slot):
            base = pl.multiple_of(ch_of(t) * CB, 8)
            pltpu.async_copy(row_hbm.at[pl.ds(base, CB)], idx.at[slot],
                             lsems[slot])
            pltpu.async_copy(oe_hbm.at[pl.ds(base, CB)], buf.at[slot],
                             lsems[slot])

        @pl.when(ch_of(0) < nchunk)
        def _():
            issue_loads(0, 0)

        nit = (maxk + 15) // 16

        @pl.loop(0, (nit + 1) // 2)
        def _(t2):
            for b in range(2):
                slot, nslot = b, 1 - b
                t = t2 * 2 + b

                @pl.when(ch_of(t + 1) < nchunk)
                def _():
                    issue_loads(t + 1, nslot)

                @pl.when(ch_of(t) < nchunk)
                def _():
                    pltpu.make_async_copy(row_hbm.at[pl.ds(0, CB)],
                                          idx.at[slot], lsems[slot]).wait()
                    pltpu.make_async_copy(oe_hbm.at[pl.ds(0, CB)],
                                          buf.at[slot], lsems[slot]).wait()
                    pltpu.sync_copy(buf.at[slot], acc.at[idx.at[slot]],
                                    add=True)

        plsc.subcore_barrier()

        @pl.loop(0, (NZB + 15) // 16)
        def _(t):
            blkid = sid + t * 16

            @pl.when(blkid < NZB)
            def _():
                off = pl.multiple_of(blkid * ZB, 8)
                pltpu.sync_copy(acc.at[pl.ds(off, ZB)], zbuf)
                pltpu.sync_copy(zbuf, p_hbm.at[cid, pl.ds(off, ZB)])

    return k(out_e, row)


def _tc_post_body(h_ref, c_ref, *rest):
    p_refs = rest[:NSLAB]
    (nw0l_ref, nb0_ref, nw1_ref, nb1_ref, nw2_ref, nb2_ref,
     hout_ref, cout_ref) = rest[NSLAB:]
    acc = p_refs[0][0] + p_refs[0][1]
    for pr in p_refs[1:]:
        acc = acc + pr[0] + pr[1]
    qagg = acc[:, :64]
    ts = acc[:, 64:67]
    cnt = acc[:, 67:68]
    h = h_ref[...]
    y = _silu(_dot_t(h, nw0l_ref[...]) + qagg + nb0_ref[...])
    y = _silu(_dot_t(y, nw1_ref[...]) + nb1_ref[...])
    y = _dot_t(y, nw2_ref[...]) + nb2_ref[...]
    hout_ref[...] = h + y
    cout_ref[...] = c_ref[...] + ts / jnp.clip(cnt, 1.0, None)


def _tc_post(h, coord, ps, nw0l, nb0, nw1, nb1, nw2, nb2):
    blk = 2000
    grid = N // blk
    full = lambda shp: pl.BlockSpec(shp, lambda i: tuple(0 for _ in shp))
    return pl.pallas_call(
        _tc_post_body,
        grid=(grid,),
        in_specs=[
            pl.BlockSpec((blk, D), lambda i: (i, 0)),
            pl.BlockSpec((blk, 3), lambda i: (i, 0)),
        ] + [
            pl.BlockSpec((2, blk, OW), lambda i: (0, i, 0)) for _ in ps
        ] + [
            full(nw0l.shape), full(nb0.shape), full(nw1.shape),
            full(nb1.shape), full(nw2.shape), full(nb2.shape),
        ],
        out_specs=[
            pl.BlockSpec((blk, D), lambda i: (i, 0)),
            pl.BlockSpec((blk, 3), lambda i: (i, 0)),
        ],
        out_shape=[
            jax.ShapeDtypeStruct((N, D), jnp.float32),
            jax.ShapeDtypeStruct((N, 3), jnp.float32),
        ],
    )(h, coord, *ps, nw0l, nb0, nw1, nb1, nw2, nb2)


def kernel(h, edge_index, coord, edge_attr,
           ew0, eb0, ew1, eb1, ew2, eb2,
           nw0, nb0, nw1, nb1, nw2, nb2,
           cw0, cb0, cw1, cb1, cw2):
    row = edge_index[0]
    col = edge_index[1]
    a = ew0[:, :D]
    b = ew0[:, D:2 * D]
    wr = ew0[:, 2 * D:2 * D + 1].T
    c4 = ew0[:, 2 * D + 1:].T

    nw0l = nw0[:, :D]
    nw0r = nw0[:, D:]

    ta, tb = _tc_pre(h, coord, a, b)
    slabs = [slice(si * SL, (si + 1) * SL) for si in range(NSLAB)]
    gabs = [_sc_gather(ta, tb, edge_index[:, sl].reshape(-1)) for sl in slabs]
    oes = [_tc_edge(gab, edge_attr[sl], wr, c4,
                    eb0.reshape(1, -1), ew1, eb1.reshape(1, -1),
                    ew2, eb2.reshape(1, -1),
                    cw0, cb0.reshape(1, -1), cw1, cb1.reshape(1, -1), cw2,
                    nw0r)
           for gab, sl in zip(gabs, slabs)]
    ps = [_sc_scatter(oe, row[sl]) for oe, sl in zip(oes, slabs)]
    h_out, coord_out = _tc_post(h, coord, ps, nw0l, nb0.reshape(1, -1),
                                nw1, nb1.reshape(1, -1), nw2, nb2.reshape(1, -1))
    return (h_out, coord_out, edge_attr)


# speedup vs baseline: 1.0346x; 1.0346x over previous
import functools

import jax
import jax.numpy as jnp
from jax import lax
from jax.experimental import pallas as pl
from jax.experimental.pallas import tpu as pltpu
from jax.experimental.pallas import tpu_sc as plsc

N = 10000
E = 320000
D = 128
TW = 128
OW = 128
CB = 128
NWORK = 32
ZB = 80
NZB = N // ZB
NSLAB = 4
SL = E // NSLAB


def _silu(x):
    t = 0.5 * x
    return t * (jnp.tanh(t) + 1.0)


def _silu_b(x):
    xb = x.astype(jnp.bfloat16)
    return xb * jax.nn.sigmoid(xb)


def _dot_t(x, w):
    return lax.dot_general(x, w, (((1,), (1,)), ((), ())),
                           preferred_element_type=jnp.float32)


def _dot_t_bf(x, w):
    return lax.dot_general(x.astype(jnp.bfloat16), w.astype(jnp.bfloat16),
                           (((1,), (1,)), ((), ())),
                           preferred_element_type=jnp.float32)


def _tc_pre_body(h_ref, c_ref, a_ref, b_ref, ta_ref, tb_ref):
    h = h_ref[...]
    cd = c_ref[...]
    pad = jnp.zeros((h.shape[0], TW - 67), jnp.float32)
    ha = _dot_t(h, a_ref[...])
    hb = _dot_t(h, b_ref[...])
    ta_ref[...] = jnp.concatenate([ha, cd, pad], axis=1)
    tb_ref[...] = jnp.concatenate([hb, -cd, pad], axis=1)


def _tc_pre(h, coord, a, b):
    blk = 2000
    grid = N // blk
    return pl.pallas_call(
        _tc_pre_body,
        grid=(grid,),
        in_specs=[
            pl.BlockSpec((blk, D), lambda i: (i, 0)),
            pl.BlockSpec((blk, 3), lambda i: (i, 0)),
            pl.BlockSpec((64, D), lambda i: (0, 0)),
            pl.BlockSpec((64, D), lambda i: (0, 0)),
        ],
        out_specs=[
            pl.BlockSpec((blk, TW), lambda i: (i, 0)),
            pl.BlockSpec((blk, TW), lambda i: (i, 0)),
        ],
        out_shape=[
            jax.ShapeDtypeStruct((N, TW), jnp.float32),
            jax.ShapeDtypeStruct((N, TW), jnp.float32),
        ],
    )(h, coord, a, b)


GCB = 128
GW = 128


def _sc_gather(ta, tb, ei, base, ne):
    mesh = plsc.VectorSubcoreMesh(core_axis_name="c", subcore_axis_name="s")

    nchunk = ne // GCB
    nit = (nchunk + NWORK - 1) // NWORK + 2

    @functools.partial(
        pl.kernel,
        out_type=jax.ShapeDtypeStruct((2, ne, GW), jnp.float32),
        mesh=mesh,
        compiler_params=pltpu.CompilerParams(use_tc_tiling_on_sc=True),
        scratch_types=[
            pltpu.VMEM((2, 2, GCB), jnp.int32),
            pltpu.VMEM((2, 2, GCB, TW), jnp.float32),
            pltpu.SemaphoreType.DMA,
            pltpu.SemaphoreType.DMA,
            pltpu.SemaphoreType.DMA,
            pltpu.SemaphoreType.DMA,
            pltpu.SemaphoreType.DMA,
            pltpu.SemaphoreType.DMA,
        ],
    )
    def k(ta_hbm, tb_hbm, ei_hbm, gab_hbm,
          idx, buf, isem0, isem1, gsem0, gsem1, wsem0, wsem1):
        cid = lax.axis_index("c")
        sid = lax.axis_index("s")
        wid = sid * 2 + cid
        isems = (isem0, isem1)
        gsems = (gsem0, gsem1)
        wsems = (wsem0, wsem1)

        def chunk_of(t):
            return wid + t * NWORK

        def valid(t):
            return jnp.logical_and(t >= 0, chunk_of(t) < nchunk)

        def issue_idx(t, slot):
            off = pl.multiple_of(base + chunk_of(t) * GCB, 128)
            pltpu.async_copy(ei_hbm.at[0, pl.ds(off, GCB)], idx.at[slot, 0],
                             isems[slot])
            pltpu.async_copy(ei_hbm.at[1, pl.ds(off, GCB)],
                             idx.at[slot, 1], isems[slot])

        def issue_gather(slot):
            pltpu.async_copy(ta_hbm.at[idx.at[slot, 0]], buf.at[slot, 0],
                             gsems[slot])
            pltpu.async_copy(tb_hbm.at[idx.at[slot, 1]], buf.at[slot, 1],
                             gsems[slot])

        def wait(sem, dst):
            pltpu.make_async_copy(ta_hbm.at[pl.ds(0, dst.shape[-2])], dst,
                                  sem).wait()

        @pl.when(valid(0))
        def _():
            issue_idx(0, 0)

        @pl.loop(0, (nit + 1) // 2)
        def _(t2):
            for b in range(2):
                slot, nslot = b, 1 - b
                t = t2 * 2 + b

                @pl.when(valid(t - 2))
                def _():
                    pltpu.make_async_copy(buf.at[slot, :, :, pl.ds(0, GW)],
                                          gab_hbm.at[:, pl.ds(0, GCB)],
                                          wsems[slot]).wait()

                @pl.when(valid(t))
                def _():
                    pltpu.make_async_copy(ei_hbm.at[0, pl.ds(0, GCB)],
                                          idx.at[slot, 0], isems[slot]).wait()
                    pltpu.make_async_copy(ei_hbm.at[1, pl.ds(0, GCB)],
                                          idx.at[slot, 1], isems[slot]).wait()
                    issue_gather(slot)

                @pl.when(valid(t - 1))
                def _():
                    wait(gsems[nslot], buf.at[nslot, 0])
                    wait(gsems[nslot], buf.at[nslot, 1])
                    base = pl.multiple_of(chunk_of(t - 1) * GCB, 8)
                    pltpu.async_copy(buf.at[nslot, :, :, pl.ds(0, GW)],
                                     gab_hbm.at[:, pl.ds(base, GCB)],
                                     wsems[nslot])

                @pl.when(valid(t + 1))
                def _():
                    issue_idx(t + 1, nslot)

    return k(ta, tb, ei)


def _tc_edge_body(ga_ref, gb_ref, ea_ref,
                  wr_ref, c4_ref, eb0_ref, ew1_ref, eb1_ref, ew2_ref, eb2_ref,
                  cw0_ref, cb0_ref, cw1_ref, cb1_ref, cw2_ref, nw0r_ref,
                  out_ref):
    s = ga_ref[0] + gb_ref[0]
    z0p = s[:, :64]
    cdiff = s[:, 64:67]
    radial = jnp.sum(cdiff * cdiff, axis=1, keepdims=True)
    ea = ea_ref[...]
    z0 = z0p + radial * wr_ref[...] + jnp.dot(
        ea, c4_ref[...], preferred_element_type=jnp.float32) + eb0_ref[...]
    z0 = _silu(z0)
    z1 = _silu(_dot_t_bf(z0, ew1_ref[...]) + eb1_ref[...])
    ef = _silu(_dot_t_bf(z1, ew2_ref[...]) + eb2_ref[...])
    c0 = _silu(_dot_t_bf(ef, cw0_ref[...]) + cb0_ref[...])
    c1 = _silu(_dot_t_bf(c0, cw1_ref[...]) + cb1_ref[...])
    cc = jnp.sum(c1 * cw2_ref[...], axis=1, keepdims=True)
    trans = jnp.clip(cdiff * cc, -100.0, 100.0)
    q = _dot_t_bf(ef, nw0r_ref[...])
    nb = s.shape[0]
    out_ref[:, 0:64] = q
    out_ref[:, 64:67] = trans
    out_ref[:, 67:68] = jnp.ones((nb, 1), jnp.float32)


def _tc_edge(gab, ea, eab, wr, c4, eb0, ew1, eb1, ew2, eb2,
             cw0, cb0, cw1, cb1, cw2, nw0r):
    blk = 3200
    ne = gab.shape[1]
    grid = ne // blk
    full = lambda shp: pl.BlockSpec(shp, lambda i: tuple(0 for _ in shp))
    return pl.pallas_call(
        _tc_edge_body,
        grid=(grid,),
        in_specs=[
            pl.BlockSpec((1, blk, GW), lambda i: (0, i, 0)),
            pl.BlockSpec((1, blk, GW), lambda i: (1, i, 0)),
            pl.BlockSpec((blk, 4), lambda i: (i + eab, 0)),
            full(wr.shape), full(c4.shape), full(eb0.shape),
            full(ew1.shape), full(eb1.shape), full(ew2.shape), full(eb2.shape),
            full(cw0.shape), full(cb0.shape), full(cw1.shape), full(cb1.shape),
            full(cw2.shape), full(nw0r.shape),
        ],
        out_specs=pl.BlockSpec((blk, OW), lambda i: (i, 0)),
        out_shape=jax.ShapeDtypeStruct((ne, OW), jnp.float32),
    )(gab, gab, ea, wr, c4, eb0, ew1, eb1, ew2, eb2, cw0, cb0, cw1, cb1, cw2,
      nw0r)


def _sc_scatter(out_e, ei, base):
    mesh = plsc.VectorSubcoreMesh(core_axis_name="c", subcore_axis_name="s")
    nchunk = out_e.shape[0] // CB
    maxk = (nchunk + 1) // 2

    @functools.partial(
        pl.kernel,
        out_type=jax.ShapeDtypeStruct((2, N, OW), jnp.float32),
        mesh=mesh,
        compiler_params=pltpu.CompilerParams(use_tc_tiling_on_sc=True),
        scratch_types=[
            pltpu.VMEM((2, CB), jnp.int32),
            pltpu.VMEM((2, CB, OW), jnp.float32),
            pltpu.VMEM((ZB, OW), jnp.float32),
            pltpu.VMEM_SHARED((N, OW), jnp.float32),
            pltpu.SemaphoreType.DMA,
            pltpu.SemaphoreType.DMA,
        ],
    )
    def k(oe_hbm, ei_hbm, p_hbm, idx, buf, zbuf, acc, lsem0, lsem1):
        cid = lax.axis_index("c")
        sid = lax.axis_index("s")
        lsems = (lsem0, lsem1)

        @pl.loop(0, ZB)
        def _(i):
            @pl.loop(0, OW // 16)
            def _(j):
                zbuf[i, pl.ds(j * 16, 16)] = jnp.zeros((16,), jnp.float32)

        @pl.loop(0, (NZB + 15) // 16)
        def _(t):
            blkid = sid + t * 16

            @pl.when(blkid < NZB)
            def _():
                off = pl.multiple_of(blkid * ZB, 8)
                pltpu.sync_copy(zbuf, acc.at[pl.ds(off, ZB)])

        plsc.subcore_barrier()

        def ch_of(t):
            return (sid + t * 16) * 2 + cid

        def issue_loads(t, slot):
            off = pl.multiple_of(ch_of(t) * CB, 128)
            pltpu.async_copy(ei_hbm.at[0, pl.ds(base + off, CB)],
                             idx.at[slot], lsems[slot])
            pltpu.async_copy(oe_hbm.at[pl.ds(off, CB)], buf.at[slot],
                             lsems[slot])

        @pl.when(ch_of(0) < nchunk)
        def _():
            issue_loads(0, 0)

        nit = (maxk + 15) // 16

        @pl.loop(0, (nit + 1) // 2)
        def _(t2):
            for b in range(2):
                slot, nslot = b, 1 - b
                t = t2 * 2 + b

                @pl.when(ch_of(t + 1) < nchunk)
                def _():
                    issue_loads(t + 1, nslot)

                @pl.when(ch_of(t) < nchunk)
                def _():
                    pltpu.make_async_copy(ei_hbm.at[0, pl.ds(0, CB)],
                                          idx.at[slot], lsems[slot]).wait()
                    pltpu.make_async_copy(oe_hbm.at[pl.ds(0, CB)],
                                          buf.at[slot], lsems[slot]).wait()
                    pltpu.sync_copy(buf.at[slot], acc.at[idx.at[slot]],
                                    add=True)

        plsc.subcore_barrier()

        @pl.loop(0, (NZB + 15) // 16)
        def _(t):
            blkid = sid + t * 16

            @pl.when(blkid < NZB)
            def _():
                off = pl.multiple_of(blkid * ZB, 8)
                pltpu.sync_copy(acc.at[pl.ds(off, ZB)], zbuf)
                pltpu.sync_copy(zbuf, p_hbm.at[cid, pl.ds(off, ZB)])

    return k(out_e, ei)


def _tc_post_body(h_ref, c_ref, *rest):
    p_refs = rest[:NSLAB]
    (nw0l_ref, nb0_ref, nw1_ref, nb1_ref, nw2_ref, nb2_ref,
     hout_ref, cout_ref) = rest[NSLAB:]
    acc = p_refs[0][0] + p_refs[0][1]
    for pr in p_refs[1:]:
        acc = acc + pr[0] + pr[1]
    qagg = acc[:, :64]
    ts = acc[:, 64:67]
    cnt = acc[:, 67:68]
    h = h_ref[...]
    y = _silu(_dot_t(h, nw0l_ref[...]) + qagg + nb0_ref[...])
    y = _silu(_dot_t(y, nw1_ref[...]) + nb1_ref[...])
    y = _dot_t(y, nw2_ref[...]) + nb2_ref[...]
    hout_ref[...] = h + y
    cout_ref[...] = c_ref[...] + ts / jnp.clip(cnt, 1.0, None)


def _tc_post(h, coord, ps, nw0l, nb0, nw1, nb1, nw2, nb2):
    blk = 2000
    grid = N // blk
    full = lambda shp: pl.BlockSpec(shp, lambda i: tuple(0 for _ in shp))
    return pl.pallas_call(
        _tc_post_body,
        grid=(grid,),
        in_specs=[
            pl.BlockSpec((blk, D), lambda i: (i, 0)),
            pl.BlockSpec((blk, 3), lambda i: (i, 0)),
        ] + [
            pl.BlockSpec((2, blk, OW), lambda i: (0, i, 0)) for _ in ps
        ] + [
            full(nw0l.shape), full(nb0.shape), full(nw1.shape),
            full(nb1.shape), full(nw2.shape), full(nb2.shape),
        ],
        out_specs=[
            pl.BlockSpec((blk, D), lambda i: (i, 0)),
            pl.BlockSpec((blk, 3), lambda i: (i, 0)),
        ],
        out_shape=[
            jax.ShapeDtypeStruct((N, D), jnp.float32),
            jax.ShapeDtypeStruct((N, 3), jnp.float32),
        ],
    )(h, coord, *ps, nw0l, nb0, nw1, nb1, nw2, nb2)


def kernel(h, edge_index, coord, edge_attr,
           ew0, eb0, ew1, eb1, ew2, eb2,
           nw0, nb0, nw1, nb1, nw2, nb2,
           cw0, cb0, cw1, cb1, cw2):
    a = ew0[:, :D]
    b = ew0[:, D:2 * D]
    wr = ew0[:, 2 * D:2 * D + 1].T
    c4 = ew0[:, 2 * D + 1:].T

    nw0l = nw0[:, :D]
    nw0r = nw0[:, D:]

    ta, tb = _tc_pre(h, coord, a, b)
    gabs = [_sc_gather(ta, tb, edge_index, si * SL, SL)
            for si in range(NSLAB)]
    oes = [_tc_edge(gab, edge_attr, si * SL // 3200, wr, c4,
                    eb0.reshape(1, -1), ew1, eb1.reshape(1, -1),
                    ew2, eb2.reshape(1, -1),
                    cw0, cb0.reshape(1, -1), cw1, cb1.reshape(1, -1), cw2,
                    nw0r)
           for si, gab in enumerate(gabs)]
    ps = [_sc_scatter(oe, edge_index, si * SL) for si, oe in enumerate(oes)]
    h_out, coord_out = _tc_post(h, coord, ps, nw0l, nb0.reshape(1, -1),
                                nw1, nb1.reshape(1, -1), nw2, nb2.reshape(1, -1))
    return (h_out, coord_out, edge_attr)
